# Initial kernel scaffold; baseline (speedup 1.0000x reference)
import jax
import jax.numpy as jnp
from jax.experimental import pallas as pl


def kernel(pixel_pred, pixel_gt, pixel_weight):
    raise NotImplementedError("write your pallas kernel here")


# speedup vs baseline: 158.7772x; 158.7772x over previous
import jax
import jax.numpy as jnp
from jax import lax
from jax.experimental import pallas as pl
from jax.experimental.pallas import tpu as pltpu

_B, _C, _H, _W = 8, 2, 512, 512
_N = _B * _H * _W
_ROWS = (_H * _W) // 128
_RB = 512
_GRID = (_B, _ROWS // _RB)


def _ce_block(pred_ref, gt_ref):
    p0 = pred_ref[0, 0]
    p1 = pred_ref[0, 1]
    g = gt_ref[0]
    d = p0 - p1
    sp = jnp.log(1.0 + jnp.exp(-jnp.abs(d)))
    m_minus = jnp.where(g == 0, jnp.maximum(-d, 0.0), jnp.maximum(d, 0.0))
    return sp + m_minus


def _sums_body(pred_ref, gt_ref, wt_ref, acc_ref):
    i = pl.program_id(0)
    j = pl.program_id(1)

    @pl.when((i == 0) & (j == 0))
    def _():
        acc_ref[0] = 0.0
        acc_ref[1] = 0.0
        acc_ref[2] = 0.0

    ce = _ce_block(pred_ref, gt_ref)
    w = wt_ref[0]
    acc_ref[0] += jnp.sum(ce)
    acc_ref[1] += jnp.sum(w * ce)
    acc_ref[2] += jnp.sum(jnp.where(gt_ref[0] != 0, 1.0, 0.0))


def _temp_body(pred_ref, gt_ref, wt_ref, out_ref):
    ce = _ce_block(pred_ref, gt_ref)
    out_ref[0] = jnp.where(wt_ref[0] != 0.0, 0.0, ce)


def _count_body(mid_ref, temp_ref, cnt_ref):
    i = pl.program_id(0)
    j = pl.program_id(1)

    @pl.when((i == 0) & (j == 0))
    def _():
        cnt_ref[0] = 0.0

    cnt_ref[0] += jnp.sum(jnp.where(temp_ref[0] >= mid_ref[0], 1.0, 0.0))


def _tail_body(thr_ref, temp_ref, out_ref):
    i = pl.program_id(0)
    j = pl.program_id(1)

    @pl.when((i == 0) & (j == 0))
    def _():
        out_ref[0] = 0.0
        out_ref[1] = 0.0

    t = temp_ref[0]
    gt_mask = t > thr_ref[0]
    out_ref[0] += jnp.sum(jnp.where(gt_mask, t, 0.0))
    out_ref[1] += jnp.sum(jnp.where(gt_mask, 1.0, 0.0))


_pred_spec = pl.BlockSpec((1, _C, _RB, 128), lambda i, j: (i, 0, j, 0))
_map_spec = pl.BlockSpec((1, _RB, 128), lambda i, j: (i, j, 0))
_smem_scalar = pl.BlockSpec(memory_space=pltpu.SMEM)


def _topk_sum(pred4, gt3, wt3, k_f32):
    temp = pl.pallas_call(
        _temp_body,
        grid=_GRID,
        in_specs=[_pred_spec, _map_spec, _map_spec],
        out_specs=_map_spec,
        out_shape=jax.ShapeDtypeStruct((_B, _ROWS, 128), jnp.float32),
    )(pred4, gt3, wt3)

    count_call = pl.pallas_call(
        _count_body,
        grid=_GRID,
        in_specs=[_smem_scalar, _map_spec],
        out_specs=_smem_scalar,
        out_shape=jax.ShapeDtypeStruct((1,), jnp.float32),
    )

    def cond(c):
        lo, hi = c
        return hi - lo > 1

    def body(c):
        lo, hi = c
        mid = lo + (hi - lo) // 2
        midf = lax.bitcast_convert_type(mid, jnp.float32)
        cnt = count_call(midf.reshape(1), temp)[0]
        ge = cnt >= k_f32
        return (jnp.where(ge, mid, lo), jnp.where(ge, hi, mid))

    lo0 = jnp.int32(0)
    hi0 = jnp.int32(0x7F800000)
    lo, _ = lax.while_loop(cond, body, (lo0, hi0))
    thr = lax.bitcast_convert_type(lo, jnp.float32)

    tail = pl.pallas_call(
        _tail_body,
        grid=_GRID,
        in_specs=[_smem_scalar, _map_spec],
        out_specs=_smem_scalar,
        out_shape=jax.ShapeDtypeStruct((2,), jnp.float32),
    )(thr.reshape(1), temp)
    sum_gt, cnt_gt = tail[0], tail[1]
    return sum_gt + (k_f32 - cnt_gt) * thr


def kernel(pixel_pred, pixel_gt, pixel_weight):
    pred4 = pixel_pred.reshape(_B, _C, _ROWS, 128)
    gt3 = pixel_gt.reshape(_B, _ROWS, 128)
    wt3 = pixel_weight.reshape(_B, _ROWS, 128)

    acc = pl.pallas_call(
        _sums_body,
        grid=_GRID,
        in_specs=[_pred_spec, _map_spec, _map_spec],
        out_specs=_smem_scalar,
        out_shape=jax.ShapeDtypeStruct((3,), jnp.float32),
    )(pred4, gt3, wt3)
    s_all, s_w, area = acc[0], acc[1], acc[2]

    k_f32 = jnp.minimum(3.0 * area, float(_N))
    total = lax.cond(
        3.0 * area >= float(_N),
        lambda: s_all,
        lambda: s_w + _topk_sum(pred4, gt3, wt3, k_f32),
    )
    return total / (4.0 * area)


# speedup vs baseline: 218.6903x; 1.3773x over previous
import jax
import jax.numpy as jnp
from jax import lax
from jax.experimental import pallas as pl
from jax.experimental.pallas import tpu as pltpu

_B, _C, _H, _W = 8, 2, 512, 512
_N = _B * _H * _W
_RB = 64
_GRID = (_B, _H // _RB)


def _ce_block(pred_ref, gt_ref):
    p0 = pred_ref[0, 0]
    p1 = pred_ref[0, 1]
    g = gt_ref[0, 0]
    d = p0 - p1
    sp = jnp.log(1.0 + jnp.exp(-jnp.abs(d)))
    m_minus = jnp.where(g == 0, jnp.maximum(-d, 0.0), jnp.maximum(d, 0.0))
    return sp + m_minus


def _sums_body(pred_ref, gt_ref, wt_ref, acc_ref):
    i = pl.program_id(0)
    j = pl.program_id(1)

    @pl.when((i == 0) & (j == 0))
    def _():
        acc_ref[0] = 0.0
        acc_ref[1] = 0.0
        acc_ref[2] = 0.0

    ce = _ce_block(pred_ref, gt_ref)
    w = wt_ref[0, 0]
    acc_ref[0] += jnp.sum(ce)
    acc_ref[1] += jnp.sum(w * ce)
    acc_ref[2] += jnp.sum(jnp.where(gt_ref[0, 0] != 0, 1.0, 0.0))


def _temp_body(pred_ref, gt_ref, wt_ref, out_ref):
    ce = _ce_block(pred_ref, gt_ref)
    out_ref[0, 0] = jnp.where(wt_ref[0, 0] != 0.0, 0.0, ce)


def _count_body(mid_ref, temp_ref, cnt_ref):
    i = pl.program_id(0)
    j = pl.program_id(1)

    @pl.when((i == 0) & (j == 0))
    def _():
        cnt_ref[0] = 0.0

    cnt_ref[0] += jnp.sum(jnp.where(temp_ref[0, 0] >= mid_ref[0], 1.0, 0.0))


def _tail_body(thr_ref, temp_ref, out_ref):
    i = pl.program_id(0)
    j = pl.program_id(1)

    @pl.when((i == 0) & (j == 0))
    def _():
        out_ref[0] = 0.0
        out_ref[1] = 0.0

    t = temp_ref[0, 0]
    gt_mask = t > thr_ref[0]
    out_ref[0] += jnp.sum(jnp.where(gt_mask, t, 0.0))
    out_ref[1] += jnp.sum(jnp.where(gt_mask, 1.0, 0.0))


_pred_spec = pl.BlockSpec((1, _C, _RB, _W), lambda i, j: (i, 0, j, 0))
_map_spec = pl.BlockSpec((1, 1, _RB, _W), lambda i, j: (i, 0, j, 0))
_smem_scalar = pl.BlockSpec(memory_space=pltpu.SMEM)


def _topk_sum(pred, gt, wt, k_f32):
    temp = pl.pallas_call(
        _temp_body,
        grid=_GRID,
        in_specs=[_pred_spec, _map_spec, _map_spec],
        out_specs=_map_spec,
        out_shape=jax.ShapeDtypeStruct((_B, 1, _H, _W), jnp.float32),
    )(pred, gt, wt)

    count_call = pl.pallas_call(
        _count_body,
        grid=_GRID,
        in_specs=[_smem_scalar, _map_spec],
        out_specs=_smem_scalar,
        out_shape=jax.ShapeDtypeStruct((1,), jnp.float32),
    )

    def cond(c):
        lo, hi = c
        return hi - lo > 1

    def body(c):
        lo, hi = c
        mid = lo + (hi - lo) // 2
        midf = lax.bitcast_convert_type(mid, jnp.float32)
        cnt = count_call(midf.reshape(1), temp)[0]
        ge = cnt >= k_f32
        return (jnp.where(ge, mid, lo), jnp.where(ge, hi, mid))

    lo0 = jnp.int32(0)
    hi0 = jnp.int32(0x7F800000)
    lo, _ = lax.while_loop(cond, body, (lo0, hi0))
    thr = lax.bitcast_convert_type(lo, jnp.float32)

    tail = pl.pallas_call(
        _tail_body,
        grid=_GRID,
        in_specs=[_smem_scalar, _map_spec],
        out_specs=_smem_scalar,
        out_shape=jax.ShapeDtypeStruct((2,), jnp.float32),
    )(thr.reshape(1), temp)
    sum_gt, cnt_gt = tail[0], tail[1]
    return sum_gt + (k_f32 - cnt_gt) * thr


def kernel(pixel_pred, pixel_gt, pixel_weight):
    acc = pl.pallas_call(
        _sums_body,
        grid=_GRID,
        in_specs=[_pred_spec, _map_spec, _map_spec],
        out_specs=_smem_scalar,
        out_shape=jax.ShapeDtypeStruct((3,), jnp.float32),
    )(pixel_pred, pixel_gt, pixel_weight)
    s_all, s_w, area = acc[0], acc[1], acc[2]

    k_f32 = jnp.minimum(3.0 * area, float(_N))
    total = lax.cond(
        3.0 * area >= float(_N),
        lambda: s_all,
        lambda: s_w + _topk_sum(pixel_pred, pixel_gt, pixel_weight, k_f32),
    )
    return total / (4.0 * area)


# speedup vs baseline: 437.6425x; 2.0012x over previous
import jax
import jax.numpy as jnp
from jax import lax
from jax.experimental import pallas as pl
from jax.experimental.pallas import tpu as pltpu

_B, _C, _H, _W = 8, 2, 512, 512
_N = _B * _H * _W
_RB = 256
_GRID = (_B, _H // _RB)


def _ce_block(pred_ref, gt_ref):
    p0 = pred_ref[0, 0]
    p1 = pred_ref[0, 1]
    g = gt_ref[0, 0]
    d = p0 - p1
    sp = jnp.log(1.0 + jnp.exp(-jnp.abs(d)))
    m_minus = jnp.where(g == 0, jnp.maximum(-d, 0.0), jnp.maximum(d, 0.0))
    return sp + m_minus


def _sums_body(pred_ref, gt_ref, wt_ref, acc_ref):
    i = pl.program_id(0)
    j = pl.program_id(1)

    @pl.when((i == 0) & (j == 0))
    def _():
        acc_ref[0] = 0.0
        acc_ref[1] = 0.0
        acc_ref[2] = 0.0

    ce = _ce_block(pred_ref, gt_ref)
    w = wt_ref[0, 0]
    acc_ref[0] += jnp.sum(ce)
    acc_ref[1] += jnp.sum(w * ce)
    acc_ref[2] += jnp.sum(jnp.where(gt_ref[0, 0] != 0, 1.0, 0.0))


def _temp_body(pred_ref, gt_ref, wt_ref, out_ref):
    ce = _ce_block(pred_ref, gt_ref)
    out_ref[0, 0] = jnp.where(wt_ref[0, 0] != 0.0, 0.0, ce)


def _count_body(mid_ref, temp_ref, cnt_ref):
    i = pl.program_id(0)
    j = pl.program_id(1)

    @pl.when((i == 0) & (j == 0))
    def _():
        cnt_ref[0] = 0.0

    cnt_ref[0] += jnp.sum(jnp.where(temp_ref[0, 0] >= mid_ref[0], 1.0, 0.0))


def _tail_body(thr_ref, temp_ref, out_ref):
    i = pl.program_id(0)
    j = pl.program_id(1)

    @pl.when((i == 0) & (j == 0))
    def _():
        out_ref[0] = 0.0
        out_ref[1] = 0.0

    t = temp_ref[0, 0]
    gt_mask = t > thr_ref[0]
    out_ref[0] += jnp.sum(jnp.where(gt_mask, t, 0.0))
    out_ref[1] += jnp.sum(jnp.where(gt_mask, 1.0, 0.0))


_pred_spec = pl.BlockSpec((1, _C, _RB, _W), lambda i, j: (i, 0, j, 0))
_map_spec = pl.BlockSpec((1, 1, _RB, _W), lambda i, j: (i, 0, j, 0))
_smem_scalar = pl.BlockSpec(memory_space=pltpu.SMEM)


def _topk_sum(pred, gt, wt, k_f32):
    temp = pl.pallas_call(
        _temp_body,
        grid=_GRID,
        in_specs=[_pred_spec, _map_spec, _map_spec],
        out_specs=_map_spec,
        out_shape=jax.ShapeDtypeStruct((_B, 1, _H, _W), jnp.float32),
    )(pred, gt, wt)

    count_call = pl.pallas_call(
        _count_body,
        grid=_GRID,
        in_specs=[_smem_scalar, _map_spec],
        out_specs=_smem_scalar,
        out_shape=jax.ShapeDtypeStruct((1,), jnp.float32),
    )

    def cond(c):
        lo, hi = c
        return hi - lo > 1

    def body(c):
        lo, hi = c
        mid = lo + (hi - lo) // 2
        midf = lax.bitcast_convert_type(mid, jnp.float32)
        cnt = count_call(midf.reshape(1), temp)[0]
        ge = cnt >= k_f32
        return (jnp.where(ge, mid, lo), jnp.where(ge, hi, mid))

    lo0 = jnp.int32(0)
    hi0 = jnp.int32(0x7F800000)
    lo, _ = lax.while_loop(cond, body, (lo0, hi0))
    thr = lax.bitcast_convert_type(lo, jnp.float32)

    tail = pl.pallas_call(
        _tail_body,
        grid=_GRID,
        in_specs=[_smem_scalar, _map_spec],
        out_specs=_smem_scalar,
        out_shape=jax.ShapeDtypeStruct((2,), jnp.float32),
    )(thr.reshape(1), temp)
    sum_gt, cnt_gt = tail[0], tail[1]
    return sum_gt + (k_f32 - cnt_gt) * thr


def kernel(pixel_pred, pixel_gt, pixel_weight):
    acc = pl.pallas_call(
        _sums_body,
        grid=_GRID,
        in_specs=[_pred_spec, _map_spec, _map_spec],
        out_specs=_smem_scalar,
        out_shape=jax.ShapeDtypeStruct((3,), jnp.float32),
    )(pixel_pred, pixel_gt, pixel_weight)
    s_all, s_w, area = acc[0], acc[1], acc[2]

    k_f32 = jnp.minimum(3.0 * area, float(_N))
    total = lax.cond(
        3.0 * area >= float(_N),
        lambda: s_all,
        lambda: s_w + _topk_sum(pixel_pred, pixel_gt, pixel_weight, k_f32),
    )
    return total / (4.0 * area)


# speedup vs baseline: 524.9747x; 1.1996x over previous
import jax
import jax.numpy as jnp
from jax import lax
from jax.experimental import pallas as pl
from jax.experimental.pallas import tpu as pltpu

_B, _C, _H, _W = 8, 2, 512, 512
_N = _B * _H * _W
_RB = 512
_GRID = (_B, _H // _RB)


def _ce_block(pred_ref, gt_ref):
    p0 = pred_ref[0, 0]
    p1 = pred_ref[0, 1]
    g = gt_ref[0, 0]
    d = p0 - p1
    sp = jnp.log(1.0 + jnp.exp(-jnp.abs(d)))
    m_minus = jnp.where(g == 0, jnp.maximum(-d, 0.0), jnp.maximum(d, 0.0))
    return sp + m_minus


def _sums_body(pred_ref, gt_ref, wt_ref, acc_ref):
    i = pl.program_id(0)
    j = pl.program_id(1)

    @pl.when((i == 0) & (j == 0))
    def _():
        acc_ref[0] = 0.0
        acc_ref[1] = 0.0
        acc_ref[2] = 0.0

    ce = _ce_block(pred_ref, gt_ref)
    w = wt_ref[0, 0]
    acc_ref[0] += jnp.sum(ce)
    acc_ref[1] += jnp.sum(w * ce)
    acc_ref[2] += jnp.sum(jnp.where(gt_ref[0, 0] != 0, 1.0, 0.0))


def _temp_body(pred_ref, gt_ref, wt_ref, out_ref):
    ce = _ce_block(pred_ref, gt_ref)
    out_ref[0, 0] = jnp.where(wt_ref[0, 0] != 0.0, 0.0, ce)


def _count_body(mid_ref, temp_ref, cnt_ref):
    i = pl.program_id(0)
    j = pl.program_id(1)

    @pl.when((i == 0) & (j == 0))
    def _():
        cnt_ref[0] = 0.0

    cnt_ref[0] += jnp.sum(jnp.where(temp_ref[0, 0] >= mid_ref[0], 1.0, 0.0))


def _tail_body(thr_ref, temp_ref, out_ref):
    i = pl.program_id(0)
    j = pl.program_id(1)

    @pl.when((i == 0) & (j == 0))
    def _():
        out_ref[0] = 0.0
        out_ref[1] = 0.0

    t = temp_ref[0, 0]
    gt_mask = t > thr_ref[0]
    out_ref[0] += jnp.sum(jnp.where(gt_mask, t, 0.0))
    out_ref[1] += jnp.sum(jnp.where(gt_mask, 1.0, 0.0))


_pred_spec = pl.BlockSpec((1, _C, _RB, _W), lambda i, j: (i, 0, j, 0))
_map_spec = pl.BlockSpec((1, 1, _RB, _W), lambda i, j: (i, 0, j, 0))
_smem_scalar = pl.BlockSpec(memory_space=pltpu.SMEM)


def _topk_sum(pred, gt, wt, k_f32):
    temp = pl.pallas_call(
        _temp_body,
        grid=_GRID,
        in_specs=[_pred_spec, _map_spec, _map_spec],
        out_specs=_map_spec,
        out_shape=jax.ShapeDtypeStruct((_B, 1, _H, _W), jnp.float32),
    )(pred, gt, wt)

    count_call = pl.pallas_call(
        _count_body,
        grid=_GRID,
        in_specs=[_smem_scalar, _map_spec],
        out_specs=_smem_scalar,
        out_shape=jax.ShapeDtypeStruct((1,), jnp.float32),
    )

    def cond(c):
        lo, hi = c
        return hi - lo > 1

    def body(c):
        lo, hi = c
        mid = lo + (hi - lo) // 2
        midf = lax.bitcast_convert_type(mid, jnp.float32)
        cnt = count_call(midf.reshape(1), temp)[0]
        ge = cnt >= k_f32
        return (jnp.where(ge, mid, lo), jnp.where(ge, hi, mid))

    lo0 = jnp.int32(0)
    hi0 = jnp.int32(0x7F800000)
    lo, _ = lax.while_loop(cond, body, (lo0, hi0))
    thr = lax.bitcast_convert_type(lo, jnp.float32)

    tail = pl.pallas_call(
        _tail_body,
        grid=_GRID,
        in_specs=[_smem_scalar, _map_spec],
        out_specs=_smem_scalar,
        out_shape=jax.ShapeDtypeStruct((2,), jnp.float32),
    )(thr.reshape(1), temp)
    sum_gt, cnt_gt = tail[0], tail[1]
    return sum_gt + (k_f32 - cnt_gt) * thr


def kernel(pixel_pred, pixel_gt, pixel_weight):
    acc = pl.pallas_call(
        _sums_body,
        grid=_GRID,
        in_specs=[_pred_spec, _map_spec, _map_spec],
        out_specs=_smem_scalar,
        out_shape=jax.ShapeDtypeStruct((3,), jnp.float32),
    )(pixel_pred, pixel_gt, pixel_weight)
    s_all, s_w, area = acc[0], acc[1], acc[2]

    k_f32 = jnp.minimum(3.0 * area, float(_N))
    total = lax.cond(
        3.0 * area >= float(_N),
        lambda: s_all,
        lambda: s_w + _topk_sum(pixel_pred, pixel_gt, pixel_weight, k_f32),
    )
    return total / (4.0 * area)
